# Initial kernel scaffold; baseline (speedup 1.0000x reference)
import functools

import jax
import jax.numpy as jnp
from jax import lax
from jax.experimental import pallas as pl
from jax.experimental.pallas import tpu as pltpu
from jax.experimental.pallas import tpu_sc as plsc

N_NODES = 10000
N_EDGES = 320000
NEDGEIN = 16
K = 4
NINP = 128
NOUT1 = 64
NOUT2 = 64

NC = 2
NS = 16
NW = NC * NS
EPW = N_EDGES // NW
CH = 80
NCHUNK = EPW // CH
RPT = N_NODES // NS
ZR = 125

_HP = lax.Precision.HIGHEST


def _edge_mlp_body(ea_ref, w1_ref, w2_ref, w3_ref, w4_ref, out_ref):
    ea = ea_ref[...]
    a = jnp.maximum(jnp.dot(ea, w1_ref[...], precision=_HP), 0.0)
    b = jnp.maximum(jnp.dot(ea, w2_ref[...], precision=_HP), 0.0) * \
        jnp.maximum(jnp.dot(ea, w3_ref[...], precision=_HP), 0.0)
    cat = jnp.concatenate([a, b], axis=1)
    out_ref[...] = jnp.maximum(jnp.dot(cat, w4_ref[...], precision=_HP), 0.0)


def _edge_mlp(edge_attr, W1, W2, W3, W4):
    BE = 8000
    return pl.pallas_call(
        _edge_mlp_body,
        grid=(N_EDGES // BE,),
        in_specs=[
            pl.BlockSpec((BE, NEDGEIN), lambda i: (i, 0)),
            pl.BlockSpec((NEDGEIN, 2 * NEDGEIN), lambda i: (0, 0)),
            pl.BlockSpec((NEDGEIN, 2 * NEDGEIN), lambda i: (0, 0)),
            pl.BlockSpec((NEDGEIN, 2 * NEDGEIN), lambda i: (0, 0)),
            pl.BlockSpec((4 * NEDGEIN, K), lambda i: (0, 0)),
        ],
        out_specs=pl.BlockSpec((BE, K), lambda i: (i, 0)),
        out_shape=jax.ShapeDtypeStruct((N_EDGES, K), jnp.float32),
    )(edge_attr, W1, W2, W3, W4)


def _xw_body(x_ref, w_ref, out_ref):
    out_ref[...] = jnp.dot(x_ref[...], w_ref[...], precision=_HP)


def _xw(x, Wcat):
    BN = 2000
    return pl.pallas_call(
        _xw_body,
        grid=(N_NODES // BN,),
        in_specs=[
            pl.BlockSpec((BN, NINP), lambda i: (i, 0)),
            pl.BlockSpec((NINP, K * NOUT1), lambda i: (0, 0)),
        ],
        out_specs=pl.BlockSpec((BN, K * NOUT1), lambda i: (i, 0)),
        out_shape=jax.ShapeDtypeStruct((N_NODES, K * NOUT1), jnp.float32),
    )(x, Wcat)


def _sc_segsum(xw, ea_flat, src, dst):
    mesh = plsc.VectorSubcoreMesh(core_axis_name="c", subcore_axis_name="s")

    @functools.partial(
        pl.kernel,
        mesh=mesh,
        out_type=jax.ShapeDtypeStruct((NC, N_NODES, NOUT1), jnp.float32),
        scratch_types=[
            pltpu.VMEM((CH,), jnp.int32),
            pltpu.VMEM((CH,), jnp.int32),
            pltpu.VMEM((CH * K,), jnp.float32),
            pltpu.VMEM((CH, K * NOUT1), jnp.float32),
            pltpu.VMEM((CH, NOUT1), jnp.float32),
            pltpu.VMEM((ZR, NOUT1), jnp.float32),
            pltpu.VMEM_SHARED((N_NODES, NOUT1), jnp.float32),
            pltpu.SemaphoreType.DMA,
        ],
    )
    def k(xw_hbm, ea_hbm, src_hbm, dst_hbm, out_hbm,
          srcv, dstv, eav, rows, msg, zbuf, accum, sem):
        cid = lax.axis_index("c")
        sid = lax.axis_index("s")
        wid = sid * NC + cid

        def zrow(r, _):
            for j in range(NOUT1 // 16):
                zbuf[r, pl.ds(j * 16, 16)] = jnp.zeros((16,), jnp.float32)
            return 0
        lax.fori_loop(0, ZR, zrow, 0)
        for z in range(RPT // ZR):
            pltpu.sync_copy(zbuf, accum.at[pl.ds(sid * RPT + z * ZR, ZR), :])
        plsc.subcore_barrier()

        def chunk(kk, _):
            base = wid * EPW + kk * CH
            pltpu.sync_copy(src_hbm.at[pl.ds(base, CH)], srcv)
            pltpu.sync_copy(dst_hbm.at[pl.ds(base, CH)], dstv)
            pltpu.sync_copy(ea_hbm.at[pl.ds(base * K, CH * K)], eav)
            pltpu.async_copy(xw_hbm.at[srcv], rows, sem).wait()

            def edge(e, _):
                a0 = eav[e * K + 0]
                a1 = eav[e * K + 1]
                a2 = eav[e * K + 2]
                a3 = eav[e * K + 3]
                for j in range(NOUT1 // 16):
                    v = a0 * rows[e, pl.ds(j * 16, 16)]
                    v = v + a1 * rows[e, pl.ds(NOUT1 + j * 16, 16)]
                    v = v + a2 * rows[e, pl.ds(2 * NOUT1 + j * 16, 16)]
                    v = v + a3 * rows[e, pl.ds(3 * NOUT1 + j * 16, 16)]
                    msg[e, pl.ds(j * 16, 16)] = v
                return 0
            lax.fori_loop(0, CH, edge, 0)

            pltpu.sync_copy(msg, accum.at[dstv], add=True)
            return 0
        lax.fori_loop(0, NCHUNK, chunk, 0)
        plsc.subcore_barrier()

        pltpu.sync_copy(accum.at[pl.ds(sid * RPT, RPT), :],
                        out_hbm.at[cid, pl.ds(sid * RPT, RPT), :])

    return k(xw, ea_flat, src, dst)


def _combine_body(p_ref, x_ref, cb_ref, w11_ref, b11_ref, w12_ref, b12_ref, out_ref):
    s = p_ref[0] + p_ref[1] + cb_ref[...]
    spect = jnp.maximum(s, 0.0)
    xx = x_ref[...]
    e1 = jnp.tanh(jnp.dot(xx, w11_ref[...], precision=_HP) + b11_ref[...])
    e2 = jnp.tanh(jnp.dot(xx, w12_ref[...], precision=_HP) + b12_ref[...])
    out_ref[...] = jnp.concatenate([spect, e1 * e2], axis=1)


def _combine(partials, x, convb, W11, b11, W12, b12):
    BN = 2000
    return pl.pallas_call(
        _combine_body,
        grid=(N_NODES // BN,),
        in_specs=[
            pl.BlockSpec((NC, BN, NOUT1), lambda i: (0, i, 0)),
            pl.BlockSpec((BN, NINP), lambda i: (i, 0)),
            pl.BlockSpec((1, NOUT1), lambda i: (0, 0)),
            pl.BlockSpec((NINP, NOUT2), lambda i: (0, 0)),
            pl.BlockSpec((1, NOUT2), lambda i: (0, 0)),
            pl.BlockSpec((NINP, NOUT2), lambda i: (0, 0)),
            pl.BlockSpec((1, NOUT2), lambda i: (0, 0)),
        ],
        out_specs=pl.BlockSpec((BN, NOUT1 + NOUT2), lambda i: (i, 0)),
        out_shape=jax.ShapeDtypeStruct((N_NODES, NOUT1 + NOUT2), jnp.float32),
    )(partials, x, convb.reshape(1, NOUT1), W11, b11.reshape(1, NOUT2),
      W12, b12.reshape(1, NOUT2))


def kernel(x, edge_index, edge_attr, W1, W2, W3, W4, convW, convb, W11, b11, W12, b12):
    src = edge_index[0].astype(jnp.int32)
    dst = edge_index[1].astype(jnp.int32)
    eattr = _edge_mlp(edge_attr, W1, W2, W3, W4)
    Wcat = jnp.transpose(convW, (1, 0, 2)).reshape(NINP, K * NOUT1)
    xw = _xw(x, Wcat)
    partials = _sc_segsum(xw, eattr.reshape(-1), src, dst)
    return _combine(partials, x, convb, W11, b11, W12, b12)


# speedup vs baseline: 2.4326x; 2.4326x over previous
import functools

import jax
import jax.numpy as jnp
from jax import lax
from jax.experimental import pallas as pl
from jax.experimental.pallas import tpu as pltpu
from jax.experimental.pallas import tpu_sc as plsc

N_NODES = 10000
N_EDGES = 320000
NEDGEIN = 16
K = 4
NINP = 128
NOUT1 = 64
NOUT2 = 64

NC = 2
NS = 16
NW = NC * NS
EPW = N_EDGES // NW
CH = 80
NCHUNK = EPW // CH
NPAD = 10240
ACCW = 128
RPT = NPAD // NS
ZR = 128

_HP = lax.Precision.HIGHEST


def _edge_mlp_body(ea_ref, w1_ref, w2_ref, w3_ref, w4_ref, out_ref):
    ea = ea_ref[...]
    a = jnp.maximum(jnp.dot(ea, w1_ref[...], precision=_HP), 0.0)
    b = jnp.maximum(jnp.dot(ea, w2_ref[...], precision=_HP), 0.0) * \
        jnp.maximum(jnp.dot(ea, w3_ref[...], precision=_HP), 0.0)
    cat = jnp.concatenate([a, b], axis=1)
    out_ref[...] = jnp.maximum(jnp.dot(cat, w4_ref[...], precision=_HP), 0.0)


def _edge_mlp(edge_attr, W1, W2, W3, W4):
    BE = 8000
    return pl.pallas_call(
        _edge_mlp_body,
        grid=(N_EDGES // BE,),
        in_specs=[
            pl.BlockSpec((BE, NEDGEIN), lambda i: (i, 0)),
            pl.BlockSpec((NEDGEIN, 2 * NEDGEIN), lambda i: (0, 0)),
            pl.BlockSpec((NEDGEIN, 2 * NEDGEIN), lambda i: (0, 0)),
            pl.BlockSpec((NEDGEIN, 2 * NEDGEIN), lambda i: (0, 0)),
            pl.BlockSpec((4 * NEDGEIN, K), lambda i: (0, 0)),
        ],
        out_specs=pl.BlockSpec((BE, K), lambda i: (i, 0)),
        out_shape=jax.ShapeDtypeStruct((N_EDGES, K), jnp.float32),
    )(edge_attr, W1, W2, W3, W4)


def _xw_body(x_ref, w_ref, out_ref):
    out_ref[...] = jnp.dot(x_ref[...], w_ref[...], precision=_HP)


def _xw(x, Wcat):
    BN = 2000
    return pl.pallas_call(
        _xw_body,
        grid=(N_NODES // BN,),
        in_specs=[
            pl.BlockSpec((BN, NINP), lambda i: (i, 0)),
            pl.BlockSpec((NINP, K * NOUT1), lambda i: (0, 0)),
        ],
        out_specs=pl.BlockSpec((BN, K * NOUT1), lambda i: (i, 0)),
        out_shape=jax.ShapeDtypeStruct((N_NODES, K * NOUT1), jnp.float32),
    )(x, Wcat)


def _sc_segsum(xw, ea_flat, src, dst):
    mesh = plsc.VectorSubcoreMesh(core_axis_name="c", subcore_axis_name="s")

    @functools.partial(
        pl.kernel,
        mesh=mesh,
        out_type=jax.ShapeDtypeStruct((NC, NPAD, ACCW), jnp.float32),
        scratch_types=[
            pltpu.VMEM((CH,), jnp.int32),
            pltpu.VMEM((CH,), jnp.int32),
            pltpu.VMEM((CH * K + 16,), jnp.float32),
            pltpu.VMEM((CH, K * NOUT1), jnp.float32),
            pltpu.VMEM((CH, ACCW), jnp.float32),
            pltpu.VMEM((ZR, ACCW), jnp.float32),
            pltpu.VMEM_SHARED((NPAD, ACCW), jnp.float32),
            pltpu.SemaphoreType.DMA,
        ],
    )
    def k(xw_hbm, ea_hbm, src_hbm, dst_hbm, out_hbm,
          srcv, dstv, eav, rows, msg, zbuf, accum, sem):
        cid = lax.axis_index("c")
        sid = lax.axis_index("s")
        wid = sid * NC + cid

        def zrow(r, _):
            for j in range(ACCW // 16):
                zbuf[r, pl.ds(j * 16, 16)] = jnp.zeros((16,), jnp.float32)
            return 0
        lax.fori_loop(0, ZR, zrow, 0)
        def mrow(r, _):
            for j in range(NOUT1 // 16):
                msg[r, pl.ds(NOUT1 + j * 16, 16)] = jnp.zeros((16,), jnp.float32)
            return 0
        lax.fori_loop(0, CH, mrow, 0)
        for z in range(RPT // ZR):
            pltpu.sync_copy(zbuf, accum.at[pl.ds(sid * RPT + z * ZR, ZR), :])
        plsc.subcore_barrier()

        def chunk(kk, _):
            base = wid * EPW + kk * CH
            pltpu.sync_copy(src_hbm.at[pl.ds(base, CH)], srcv)
            pltpu.sync_copy(dst_hbm.at[pl.ds(base, CH)], dstv)
            pltpu.sync_copy(ea_hbm.at[pl.ds(base * K, CH * K)],
                            eav.at[pl.ds(0, CH * K)])
            pltpu.async_copy(xw_hbm.at[srcv], rows, sem).wait()

            def edge(e, _):
                av = eav[pl.ds(e * K, 16)]
                a0, a1, a2, a3 = av[0], av[1], av[2], av[3]
                for j in range(NOUT1 // 16):
                    v = a0 * rows[e, pl.ds(j * 16, 16)]
                    v = v + a1 * rows[e, pl.ds(NOUT1 + j * 16, 16)]
                    v = v + a2 * rows[e, pl.ds(2 * NOUT1 + j * 16, 16)]
                    v = v + a3 * rows[e, pl.ds(3 * NOUT1 + j * 16, 16)]
                    msg[e, pl.ds(j * 16, 16)] = v
                return 0
            lax.fori_loop(0, CH, edge, 0)

            pltpu.sync_copy(msg, accum.at[dstv], add=True)
            return 0
        lax.fori_loop(0, NCHUNK, chunk, 0)
        plsc.subcore_barrier()

        pltpu.sync_copy(accum.at[pl.ds(sid * RPT, RPT), :],
                        out_hbm.at[cid, pl.ds(sid * RPT, RPT), :])

    return k(xw, ea_flat, src, dst)


def _combine_body(p_ref, x_ref, cb_ref, w11_ref, b11_ref, w12_ref, b12_ref, out_ref):
    s = p_ref[0, :, :NOUT1] + p_ref[1, :, :NOUT1] + cb_ref[...]
    spect = jnp.maximum(s, 0.0)
    xx = x_ref[...]
    e1 = jnp.tanh(jnp.dot(xx, w11_ref[...], precision=_HP) + b11_ref[...])
    e2 = jnp.tanh(jnp.dot(xx, w12_ref[...], precision=_HP) + b12_ref[...])
    out_ref[...] = jnp.concatenate([spect, e1 * e2], axis=1)


def _combine(partials, x, convb, W11, b11, W12, b12):
    BN = 2000
    return pl.pallas_call(
        _combine_body,
        grid=(N_NODES // BN,),
        in_specs=[
            pl.BlockSpec((NC, BN, ACCW), lambda i: (0, i, 0)),
            pl.BlockSpec((BN, NINP), lambda i: (i, 0)),
            pl.BlockSpec((1, NOUT1), lambda i: (0, 0)),
            pl.BlockSpec((NINP, NOUT2), lambda i: (0, 0)),
            pl.BlockSpec((1, NOUT2), lambda i: (0, 0)),
            pl.BlockSpec((NINP, NOUT2), lambda i: (0, 0)),
            pl.BlockSpec((1, NOUT2), lambda i: (0, 0)),
        ],
        out_specs=pl.BlockSpec((BN, NOUT1 + NOUT2), lambda i: (i, 0)),
        out_shape=jax.ShapeDtypeStruct((N_NODES, NOUT1 + NOUT2), jnp.float32),
    )(partials, x, convb.reshape(1, NOUT1), W11, b11.reshape(1, NOUT2),
      W12, b12.reshape(1, NOUT2))


def kernel(x, edge_index, edge_attr, W1, W2, W3, W4, convW, convb, W11, b11, W12, b12):
    src = edge_index[0].astype(jnp.int32)
    dst = edge_index[1].astype(jnp.int32)
    eattr = _edge_mlp(edge_attr, W1, W2, W3, W4)
    Wcat = jnp.transpose(convW, (1, 0, 2)).reshape(NINP, K * NOUT1)
    xw = _xw(x, Wcat)
    partials = _sc_segsum(xw, eattr.reshape(-1), src, dst)
    return _combine(partials, x, convb, W11, b11, W12, b12)


# speedup vs baseline: 3.8462x; 1.5811x over previous
import functools

import jax
import jax.numpy as jnp
from jax import lax
from jax.experimental import pallas as pl
from jax.experimental.pallas import tpu as pltpu
from jax.experimental.pallas import tpu_sc as plsc

N_NODES = 10000
N_EDGES = 320000
NEDGEIN = 16
K = 4
NINP = 128
NOUT1 = 64
NOUT2 = 64

NC = 2
NS = 16
NW = NC * NS
EPW = N_EDGES // NW
CH = 80
NCHUNK = EPW // CH
NPAD = 10240
ACCW = 128
RPT = NPAD // NS
ZR = 128


_DN1 = (((0,), (1,)), ((), ()))
_DN2 = (((0,), (0,)), ((), ()))


def _edge_mlp_body(ea_ref, w1_ref, w2_ref, w3_ref, w4a_ref, w4b_ref, out_ref):
    ea = ea_ref[...]
    aT = jnp.maximum(lax.dot_general(w1_ref[...], ea, _DN1), 0.0)
    bT = jnp.maximum(lax.dot_general(w2_ref[...], ea, _DN1), 0.0) * \
        jnp.maximum(lax.dot_general(w3_ref[...], ea, _DN1), 0.0)
    resT = lax.dot_general(w4a_ref[...], aT, _DN2) + \
        lax.dot_general(w4b_ref[...], bT, _DN2)
    out_ref[...] = jnp.maximum(resT, 0.0)


def _edge_mlp(edge_attr, W1, W2, W3, W4):
    BE = 16000
    W4a = W4[:2 * NEDGEIN]
    W4b = W4[2 * NEDGEIN:]
    return pl.pallas_call(
        _edge_mlp_body,
        grid=(N_EDGES // BE,),
        in_specs=[
            pl.BlockSpec((BE, NEDGEIN), lambda i: (i, 0)),
            pl.BlockSpec((NEDGEIN, 2 * NEDGEIN), lambda i: (0, 0)),
            pl.BlockSpec((NEDGEIN, 2 * NEDGEIN), lambda i: (0, 0)),
            pl.BlockSpec((NEDGEIN, 2 * NEDGEIN), lambda i: (0, 0)),
            pl.BlockSpec((2 * NEDGEIN, K), lambda i: (0, 0)),
            pl.BlockSpec((2 * NEDGEIN, K), lambda i: (0, 0)),
        ],
        out_specs=pl.BlockSpec((K, BE), lambda i: (0, i)),
        out_shape=jax.ShapeDtypeStruct((K, N_EDGES), jnp.float32),
    )(edge_attr, W1, W2, W3, W4a, W4b)


def _xw_body(x_ref, w_ref, out_ref):
    out_ref[...] = jnp.dot(x_ref[...], w_ref[...])


def _xw(x, Wcat):
    BN = 2000
    return pl.pallas_call(
        _xw_body,
        grid=(N_NODES // BN,),
        in_specs=[
            pl.BlockSpec((BN, NINP), lambda i: (i, 0)),
            pl.BlockSpec((NINP, K * NOUT1), lambda i: (0, 0)),
        ],
        out_specs=pl.BlockSpec((BN, K * NOUT1), lambda i: (i, 0)),
        out_shape=jax.ShapeDtypeStruct((N_NODES, K * NOUT1), jnp.float32),
    )(x, Wcat)


def _sc_segsum(xw, eaT, src, dst):
    mesh = plsc.VectorSubcoreMesh(core_axis_name="c", subcore_axis_name="s")

    @functools.partial(
        pl.kernel,
        mesh=mesh,
        out_type=jax.ShapeDtypeStruct((NC, NPAD, ACCW), jnp.float32),
        scratch_types=[
            pltpu.VMEM((CH,), jnp.int32),
            pltpu.VMEM((CH,), jnp.int32),
            pltpu.VMEM((CH,), jnp.float32),
            pltpu.VMEM((CH,), jnp.float32),
            pltpu.VMEM((CH,), jnp.float32),
            pltpu.VMEM((CH,), jnp.float32),
            pltpu.VMEM((CH, K * NOUT1), jnp.float32),
            pltpu.VMEM((CH, ACCW), jnp.float32),
            pltpu.VMEM((ZR, ACCW), jnp.float32),
            pltpu.VMEM_SHARED((NPAD, ACCW), jnp.float32),
            pltpu.SemaphoreType.DMA,
        ],
    )
    def k(xw_hbm, ea_hbm, src_hbm, dst_hbm, out_hbm,
          srcv, dstv, ea0, ea1, ea2, ea3, rows, msg, zbuf, accum, sem):
        cid = lax.axis_index("c")
        sid = lax.axis_index("s")
        wid = sid * NC + cid

        def zrow(r, _):
            for j in range(ACCW // 16):
                zbuf[r, pl.ds(j * 16, 16)] = jnp.zeros((16,), jnp.float32)
            return 0
        lax.fori_loop(0, ZR, zrow, 0)
        def mrow(r, _):
            for j in range(NOUT1 // 16):
                msg[r, pl.ds(NOUT1 + j * 16, 16)] = jnp.zeros((16,), jnp.float32)
            return 0
        lax.fori_loop(0, CH, mrow, 0)
        for z in range(RPT // ZR):
            pltpu.sync_copy(zbuf, accum.at[pl.ds(sid * RPT + z * ZR, ZR), :])
        plsc.subcore_barrier()

        def chunk(kk, _):
            base = wid * EPW + kk * CH
            pltpu.sync_copy(src_hbm.at[pl.ds(base, CH)], srcv)
            pltpu.sync_copy(dst_hbm.at[pl.ds(base, CH)], dstv)
            pltpu.sync_copy(ea_hbm.at[pl.ds(base, CH)], ea0)
            pltpu.sync_copy(ea_hbm.at[pl.ds(N_EDGES + base, CH)], ea1)
            pltpu.sync_copy(ea_hbm.at[pl.ds(2 * N_EDGES + base, CH)], ea2)
            pltpu.sync_copy(ea_hbm.at[pl.ds(3 * N_EDGES + base, CH)], ea3)
            pltpu.async_copy(xw_hbm.at[srcv], rows, sem).wait()

            def grp(g, _):
                e0 = g * 16
                a0v = ea0[pl.ds(e0, 16)]
                a1v = ea1[pl.ds(e0, 16)]
                a2v = ea2[pl.ds(e0, 16)]
                a3v = ea3[pl.ds(e0, 16)]
                for jj in range(16):
                    e = e0 + jj
                    for j in range(NOUT1 // 16):
                        v = a0v[jj] * rows[e, pl.ds(j * 16, 16)]
                        v = v + a1v[jj] * rows[e, pl.ds(NOUT1 + j * 16, 16)]
                        v = v + a2v[jj] * rows[e, pl.ds(2 * NOUT1 + j * 16, 16)]
                        v = v + a3v[jj] * rows[e, pl.ds(3 * NOUT1 + j * 16, 16)]
                        msg[e, pl.ds(j * 16, 16)] = v
                return 0
            lax.fori_loop(0, CH // 16, grp, 0)

            pltpu.sync_copy(msg, accum.at[dstv], add=True)
            return 0
        lax.fori_loop(0, NCHUNK, chunk, 0)
        plsc.subcore_barrier()

        pltpu.sync_copy(accum.at[pl.ds(sid * RPT, RPT), :],
                        out_hbm.at[cid, pl.ds(sid * RPT, RPT), :])

    return k(xw, eaT.reshape(K * N_EDGES), src, dst)


def _combine_body(p_ref, x_ref, cb_ref, w11_ref, b11_ref, w12_ref, b12_ref, out_ref):
    s = p_ref[0, :, :NOUT1] + p_ref[1, :, :NOUT1] + cb_ref[...]
    spect = jnp.maximum(s, 0.0)
    xx = x_ref[...]
    e1 = jnp.tanh(jnp.dot(xx, w11_ref[...]) + b11_ref[...])
    e2 = jnp.tanh(jnp.dot(xx, w12_ref[...]) + b12_ref[...])
    out_ref[...] = jnp.concatenate([spect, e1 * e2], axis=1)


def _combine(partials, x, convb, W11, b11, W12, b12):
    BN = 2000
    return pl.pallas_call(
        _combine_body,
        grid=(N_NODES // BN,),
        in_specs=[
            pl.BlockSpec((NC, BN, ACCW), lambda i: (0, i, 0)),
            pl.BlockSpec((BN, NINP), lambda i: (i, 0)),
            pl.BlockSpec((1, NOUT1), lambda i: (0, 0)),
            pl.BlockSpec((NINP, NOUT2), lambda i: (0, 0)),
            pl.BlockSpec((1, NOUT2), lambda i: (0, 0)),
            pl.BlockSpec((NINP, NOUT2), lambda i: (0, 0)),
            pl.BlockSpec((1, NOUT2), lambda i: (0, 0)),
        ],
        out_specs=pl.BlockSpec((BN, NOUT1 + NOUT2), lambda i: (i, 0)),
        out_shape=jax.ShapeDtypeStruct((N_NODES, NOUT1 + NOUT2), jnp.float32),
    )(partials, x, convb.reshape(1, NOUT1), W11, b11.reshape(1, NOUT2),
      W12, b12.reshape(1, NOUT2))


def kernel(x, edge_index, edge_attr, W1, W2, W3, W4, convW, convb, W11, b11, W12, b12):
    src = edge_index[0].astype(jnp.int32)
    dst = edge_index[1].astype(jnp.int32)
    eattrT = _edge_mlp(edge_attr, W1, W2, W3, W4)
    Wcat = jnp.transpose(convW, (1, 0, 2)).reshape(NINP, K * NOUT1)
    xw = _xw(x, Wcat)
    partials = _sc_segsum(xw, eattrT, src, dst)
    return _combine(partials, x, convb, W11, b11, W12, b12)


# speedup vs baseline: 4.7253x; 1.2286x over previous
import functools

import jax
import jax.numpy as jnp
from jax import lax
from jax.experimental import pallas as pl
from jax.experimental.pallas import tpu as pltpu
from jax.experimental.pallas import tpu_sc as plsc

N_NODES = 10000
N_EDGES = 320000
NEDGEIN = 16
K = 4
NINP = 128
NOUT1 = 64
NOUT2 = 64

NC = 2
NS = 16
NW = NC * NS
EPW = N_EDGES // NW
CH = 40
EAPW = 48
NCHUNK = EPW // CH
NPAD = 10240
ACCW = 128
RPT = NPAD // NS
ZR = 64


_DN1 = (((0,), (1,)), ((), ()))
_DN2 = (((0,), (0,)), ((), ()))


def _edge_mlp_body(ea_ref, w1_ref, w2_ref, w3_ref, w4a_ref, w4b_ref, out_ref):
    ea = ea_ref[...]
    aT = jnp.maximum(lax.dot_general(w1_ref[...], ea, _DN1), 0.0)
    bT = jnp.maximum(lax.dot_general(w2_ref[...], ea, _DN1), 0.0) * \
        jnp.maximum(lax.dot_general(w3_ref[...], ea, _DN1), 0.0)
    resT = lax.dot_general(w4a_ref[...], aT, _DN2) + \
        lax.dot_general(w4b_ref[...], bT, _DN2)
    out_ref[...] = jnp.maximum(resT, 0.0)


def _edge_mlp(edge_attr, W1, W2, W3, W4):
    BE = 16000
    W4a = W4[:2 * NEDGEIN]
    W4b = W4[2 * NEDGEIN:]
    return pl.pallas_call(
        _edge_mlp_body,
        grid=(N_EDGES // BE,),
        in_specs=[
            pl.BlockSpec((BE, NEDGEIN), lambda i: (i, 0)),
            pl.BlockSpec((NEDGEIN, 2 * NEDGEIN), lambda i: (0, 0)),
            pl.BlockSpec((NEDGEIN, 2 * NEDGEIN), lambda i: (0, 0)),
            pl.BlockSpec((NEDGEIN, 2 * NEDGEIN), lambda i: (0, 0)),
            pl.BlockSpec((2 * NEDGEIN, K), lambda i: (0, 0)),
            pl.BlockSpec((2 * NEDGEIN, K), lambda i: (0, 0)),
        ],
        out_specs=pl.BlockSpec((K, BE), lambda i: (0, i)),
        out_shape=jax.ShapeDtypeStruct((K, N_EDGES), jnp.float32),
    )(edge_attr, W1, W2, W3, W4a, W4b)


def _xw_body(x_ref, w_ref, out_ref):
    out_ref[...] = jnp.dot(x_ref[...], w_ref[...])


def _xw(x, Wcat):
    BN = 2000
    return pl.pallas_call(
        _xw_body,
        grid=(N_NODES // BN,),
        in_specs=[
            pl.BlockSpec((BN, NINP), lambda i: (i, 0)),
            pl.BlockSpec((NINP, K * NOUT1), lambda i: (0, 0)),
        ],
        out_specs=pl.BlockSpec((BN, K * NOUT1), lambda i: (i, 0)),
        out_shape=jax.ShapeDtypeStruct((N_NODES, K * NOUT1), jnp.float32),
    )(x, Wcat)


def _sc_segsum(xw, eaT, src, dst):
    mesh = plsc.VectorSubcoreMesh(core_axis_name="c", subcore_axis_name="s")
    ea_pack = jnp.pad(
        eaT.reshape(K, NW * NCHUNK, CH), ((0, 0), (0, 0), (0, EAPW - CH))
    ).transpose(1, 0, 2).reshape(-1)

    @functools.partial(
        pl.kernel,
        mesh=mesh,
        out_type=jax.ShapeDtypeStruct((NC, NPAD, ACCW), jnp.float32),
        scratch_types=[
            pltpu.VMEM((CH,), jnp.int32),
            pltpu.VMEM((CH,), jnp.int32),
            pltpu.VMEM((CH,), jnp.int32),
            pltpu.VMEM((CH,), jnp.int32),
            pltpu.VMEM((K * EAPW,), jnp.float32),
            pltpu.VMEM((K * EAPW,), jnp.float32),
            pltpu.VMEM((CH, K * NOUT1), jnp.float32),
            pltpu.VMEM((CH, K * NOUT1), jnp.float32),
            pltpu.VMEM((CH, ACCW), jnp.float32),
            pltpu.VMEM((ZR, ACCW), jnp.float32),
            pltpu.VMEM_SHARED((NPAD, ACCW), jnp.float32),
            pltpu.SemaphoreType.DMA,
            pltpu.SemaphoreType.DMA,
            pltpu.SemaphoreType.DMA,
        ],
    )
    def k(xw_hbm, ea_hbm, src_hbm, dst_hbm, out_hbm,
          srcA, dstA, srcB, dstB, eaA, eaB, rowsA, rowsB,
          msg, zbuf, accum, semA, semB, semS):
        cid = lax.axis_index("c")
        sid = lax.axis_index("s")
        wid = sid * NC + cid

        def zrow(r, _):
            for j in range(ACCW // 16):
                zbuf[r, pl.ds(j * 16, 16)] = jnp.zeros((16,), jnp.float32)
            return 0
        lax.fori_loop(0, ZR, zrow, 0)
        def mrow(r, _):
            for j in range(NOUT1 // 16):
                msg[r, pl.ds(NOUT1 + j * 16, 16)] = jnp.zeros((16,), jnp.float32)
            return 0
        lax.fori_loop(0, CH, mrow, 0)
        for z in range(RPT // ZR):
            pltpu.sync_copy(zbuf, accum.at[pl.ds(sid * RPT + z * ZR, ZR), :])
        plsc.subcore_barrier()

        def load_small(c, srcv, dstv, eav):
            base = wid * EPW + c * CH
            h1 = pltpu.async_copy(src_hbm.at[pl.ds(base, CH)], srcv, semS)
            h2 = pltpu.async_copy(dst_hbm.at[pl.ds(base, CH)], dstv, semS)
            h3 = pltpu.async_copy(
                ea_hbm.at[pl.ds((wid * NCHUNK + c) * K * EAPW, K * EAPW)],
                eav, semS)
            h1.wait(); h2.wait(); h3.wait()

        def combine_scatter(eav, rows, dstv):
            def edge16(e0, nedge, a0v, a1v, a2v, a3v):
                for jj in range(nedge):
                    e = e0 + jj
                    for j in range(NOUT1 // 16):
                        v = a0v[jj] * rows[e, pl.ds(j * 16, 16)]
                        v = v + a1v[jj] * rows[e, pl.ds(NOUT1 + j * 16, 16)]
                        v = v + a2v[jj] * rows[e, pl.ds(2 * NOUT1 + j * 16, 16)]
                        v = v + a3v[jj] * rows[e, pl.ds(3 * NOUT1 + j * 16, 16)]
                        msg[e, pl.ds(j * 16, 16)] = v

            for g in range(CH // 16 + (1 if CH % 16 else 0)):
                e0 = g * 16
                edge16(e0, min(16, CH - e0),
                       eav[pl.ds(e0, 16)],
                       eav[pl.ds(EAPW + e0, 16)],
                       eav[pl.ds(2 * EAPW + e0, 16)],
                       eav[pl.ds(3 * EAPW + e0, 16)])
            pltpu.sync_copy(msg, accum.at[dstv], add=True)

        load_small(0, srcA, dstA, eaA)
        pltpu.async_copy(xw_hbm.at[srcA], rowsA, semA)

        def pair(t, _):
            c = 2 * t
            load_small(c + 1, srcB, dstB, eaB)
            hB = pltpu.async_copy(xw_hbm.at[srcB], rowsB, semB)
            pltpu.make_async_copy(xw_hbm.at[srcA], rowsA, semA).wait()
            combine_scatter(eaA, rowsA, dstA)
            load_small(jnp.minimum(c + 2, NCHUNK - 1), srcA, dstA, eaA)
            pltpu.async_copy(xw_hbm.at[srcA], rowsA, semA)
            hB.wait()
            combine_scatter(eaB, rowsB, dstB)
            return 0
        lax.fori_loop(0, NCHUNK // 2, pair, 0)

        pltpu.make_async_copy(xw_hbm.at[srcA], rowsA, semA).wait()
        plsc.subcore_barrier()

        pltpu.sync_copy(accum.at[pl.ds(sid * RPT, RPT), :],
                        out_hbm.at[cid, pl.ds(sid * RPT, RPT), :])

    return k(xw, ea_pack, src, dst)


def _combine_body(p_ref, x_ref, cb_ref, w11_ref, b11_ref, w12_ref, b12_ref, out_ref):
    s = p_ref[0, :, :NOUT1] + p_ref[1, :, :NOUT1] + cb_ref[...]
    spect = jnp.maximum(s, 0.0)
    xx = x_ref[...]
    e1 = jnp.tanh(jnp.dot(xx, w11_ref[...]) + b11_ref[...])
    e2 = jnp.tanh(jnp.dot(xx, w12_ref[...]) + b12_ref[...])
    out_ref[...] = jnp.concatenate([spect, e1 * e2], axis=1)


def _combine(partials, x, convb, W11, b11, W12, b12):
    BN = 2000
    return pl.pallas_call(
        _combine_body,
        grid=(N_NODES // BN,),
        in_specs=[
            pl.BlockSpec((NC, BN, ACCW), lambda i: (0, i, 0)),
            pl.BlockSpec((BN, NINP), lambda i: (i, 0)),
            pl.BlockSpec((1, NOUT1), lambda i: (0, 0)),
            pl.BlockSpec((NINP, NOUT2), lambda i: (0, 0)),
            pl.BlockSpec((1, NOUT2), lambda i: (0, 0)),
            pl.BlockSpec((NINP, NOUT2), lambda i: (0, 0)),
            pl.BlockSpec((1, NOUT2), lambda i: (0, 0)),
        ],
        out_specs=pl.BlockSpec((BN, NOUT1 + NOUT2), lambda i: (i, 0)),
        out_shape=jax.ShapeDtypeStruct((N_NODES, NOUT1 + NOUT2), jnp.float32),
    )(partials, x, convb.reshape(1, NOUT1), W11, b11.reshape(1, NOUT2),
      W12, b12.reshape(1, NOUT2))


def kernel(x, edge_index, edge_attr, W1, W2, W3, W4, convW, convb, W11, b11, W12, b12):
    src = edge_index[0].astype(jnp.int32)
    dst = edge_index[1].astype(jnp.int32)
    eattrT = _edge_mlp(edge_attr, W1, W2, W3, W4)
    Wcat = jnp.transpose(convW, (1, 0, 2)).reshape(NINP, K * NOUT1)
    xw = _xw(x, Wcat)
    partials = _sc_segsum(xw, eattrT, src, dst)
    return _combine(partials, x, convb, W11, b11, W12, b12)
